# baseline (speedup 1.0000x reference)
import functools

import jax
import jax.numpy as jnp
from jax import lax
from jax.experimental import pallas as pl
from jax.experimental.pallas import tpu as pltpu
from jax.experimental.pallas import tpu_sc as plsc

B = 8
N = 10000
NREL = 201
NRELT = NREL + 1
D = 128
ATTN = 64
L = 3
E = 320000

_ROWS = B * N
NC = 2
NS = 16
NW = NC * NS
NPASS = 4
PR = 10240
OUT_ROWS = NC * NPASS * PR
TPR = PR // NS
C = 2000
NCHUNK = E // (NS * C)
HB = 96
HS = 80
GW = D + ATTN
GP = 2 * D

_GDN = lax.GatherDimensionNumbers(
    offset_dims=(), collapsed_slice_dims=(0,), start_index_map=(0,))


def _dyn_gather(x, idx):
    return lax.gather(x, idx[:, None], _GDN, (1,),
                      mode=lax.GatherScatterMode.PROMISE_IN_BOUNDS)


def _prefix16(x, iota):
    for sh in (1, 2, 4, 8):
        shifted = _dyn_gather(x, jnp.maximum(iota - sh, 0))
        x = x + jnp.where(iota >= sh, shifted, 0)
    return x


def _sum16(x, iota):
    for sh in (1, 2, 4, 8):
        x = x + _dyn_gather(x, iota ^ sh)
    return x


_NB = E // NW // HS


def _score_kernel_body(tab, rtab, aqt, a2t, relb, gidx, msgout,
                       rows0, rows1, msg, rtab_v, aq_v, a2_v,
                       relbv, srcv, sem0, sem1):
    c = lax.axis_index("c")
    s = lax.axis_index("s")
    iota = lax.iota(jnp.int32, 16)

    pltpu.sync_copy(rtab, rtab_v)
    pltpu.sync_copy(aqt, aq_v)
    pltpu.sync_copy(a2t, a2_v)
    w = s * NC + c
    base_e = w * (E // NW)
    pltpu.sync_copy(relb.at[pl.ds(base_e, E // NW)], relbv)
    pltpu.sync_copy(gidx.at[pl.ds(base_e, E // NW)], srcv)

    def compute(sb, rows):
        def one_group(g, _):
            rb16 = relbv[pl.ds(sb * HS + 16 * g, 16)]
            rel16 = rb16 >> 3
            b16 = rb16 & 7
            for l in range(16):
                row = 16 * g + l
                rl = rel16[l]
                bb = b16[l]
                acc = jnp.zeros((16,), jnp.float32)
                for k in range(ATTN // 16):
                    t = (rows[row, pl.ds(D + 16 * k, 16)]
                         + rtab_v[rl, pl.ds(D + 16 * k, 16)]
                         + aq_v[bb, pl.ds(16 * k, 16)])
                    acc = acc + jnp.maximum(t, 0.0) * a2_v[pl.ds(16 * k, 16)]
                tot = _sum16(acc, iota)
                sv = 1.0 / (1.0 + jnp.exp(-tot))
                for k in range(D // 16):
                    msg[row, pl.ds(16 * k, 16)] = (
                        rows[row, pl.ds(16 * k, 16)]
                        + rtab_v[rl, pl.ds(16 * k, 16)]) * sv
            return 0
        lax.fori_loop(0, HS // 16, one_group, 0)
        pltpu.sync_copy(msg, msgout.at[pl.ds(base_e + sb * HS, HS)])

    def issue(sb, rows, sem):
        pltpu.async_copy(tab.at[srcv.at[pl.ds(sb * HS, HS)]], rows, sem)

    def drain(rows, sem):
        pltpu.make_async_copy(tab.at[pl.ds(0, HS)], rows, sem).wait()

    issue(0, rows0, sem0)

    def pair(q, _):
        b0 = 2 * q
        issue(b0 + 1, rows1, sem1)
        drain(rows0, sem0)
        compute(b0, rows0)
        issue(b0 + 2, rows0, sem0)
        drain(rows1, sem1)
        compute(b0 + 1, rows1)
        return 0
    lax.fori_loop(0, (_NB - 1) // 2, pair, 0)
    drain(rows0, sem0)
    compute(_NB - 1, rows0)


def _make_score_kernel():
    mesh = plsc.VectorSubcoreMesh(core_axis_name="c", subcore_axis_name="s")
    scratch = [
        pltpu.VMEM((HS, GP), jnp.float32),
        pltpu.VMEM((HS, GP), jnp.float32),
        pltpu.VMEM((HS, D), jnp.float32),
        pltpu.VMEM((NRELT, GW), jnp.float32),
        pltpu.VMEM((B, ATTN), jnp.float32),
        pltpu.VMEM((ATTN,), jnp.float32),
        pltpu.VMEM((E // NW,), jnp.int32),
        pltpu.VMEM((E // NW,), jnp.int32),
        pltpu.SemaphoreType.DMA,
        pltpu.SemaphoreType.DMA,
    ]
    return pl.kernel(
        _score_kernel_body,
        out_type=jax.ShapeDtypeStruct((E, D), jnp.float32),
        mesh=mesh,
        scratch_types=scratch,
    )


_score_kernel = _make_score_kernel()


def _scatter_kernel_body(tab, gidx, dst, out,
                         acc, dstv, gidxv, hitg, hitd, gq, gq1, dq,
                         rows, rows1, zrow, obuf, gsem, gsem1):
    c = lax.axis_index("c")
    s = lax.axis_index("s")
    iota = lax.iota(jnp.int32, 16)
    zf16 = jnp.zeros((16,), jnp.float32)
    zi16 = jnp.zeros((16,), jnp.int32)

    def _zr(i, _):
        for k in range(D // 16):
            zrow[i, pl.ds(16 * k, 16)] = zf16
        return 0
    lax.fori_loop(0, HB, _zr, 0)

    def _zh(k, _):
        hitg[pl.ds(16 * k, 16)] = zi16
        hitd[pl.ds(16 * k, 16)] = zi16
        return 0
    lax.fori_loop(0, (C + 2 * HB) // 16, _zh, 0)

    def one_pass(p, _):
        lo = (c * NPASS + p) * PR
        hi = lo + PR

        base = s * TPR
        for t in range(TPR // HB):
            pltpu.sync_copy(zrow, acc.at[pl.ds(base + t * HB, HB)])
        rem = TPR % HB
        if rem:
            pltpu.sync_copy(zrow.at[pl.ds(0, rem)],
                            acc.at[pl.ds(base + (TPR // HB) * HB, rem)])
        plsc.subcore_barrier()

        def one_chunk(k, _):
            ebase = s * (NCHUNK * C) + k * C
            pltpu.sync_copy(dst.at[pl.ds(ebase, C)], dstv)
            pltpu.sync_copy(gidx.at[pl.ds(ebase, C)], gidxv)
            obuf[0] = 0

            def fstep(g, _):
                d16 = dstv[pl.ds(16 * g, 16)]
                m = (d16 >= lo) & (d16 < hi)
                mi = jnp.where(m, 1, 0)
                cs = _prefix16(mi, iota)
                cnt0 = cs[15]

                @pl.when(cnt0 > 0)
                def _():
                    g16 = gidxv[pl.ds(16 * g, 16)]
                    tgt = iota + 1
                    inv = jnp.zeros((16,), jnp.int32)
                    for step in (8, 4, 2, 1):
                        cand = inv + step
                        probe = _dyn_gather(cs, cand - 1)
                        inv = jnp.where(probe < tgt, cand, inv)
                    gc = _dyn_gather(g16, jnp.minimum(inv, 15))
                    dc = _dyn_gather(d16, jnp.minimum(inv, 15))
                    off0 = obuf[0]
                    r = off0 & 15
                    a = off0 - r
                    r16 = jnp.full((16,), r, jnp.int32)
                    perm = (iota - r16) & 15
                    rotg = _dyn_gather(gc, perm)
                    rotd = _dyn_gather(dc, perm)
                    sel0 = iota >= r16
                    b0 = hitg[pl.ds(a, 16)]
                    hitg[pl.ds(a, 16)] = jnp.where(sel0, rotg, b0)
                    b0d = hitd[pl.ds(a, 16)]
                    hitd[pl.ds(a, 16)] = jnp.where(sel0, rotd, b0d)
                    b1 = hitg[pl.ds(a + 16, 16)]
                    hitg[pl.ds(a + 16, 16)] = jnp.where(sel0, b1, rotg)
                    b1d = hitd[pl.ds(a + 16, 16)]
                    hitd[pl.ds(a + 16, 16)] = jnp.where(sel0, b1d, rotd)
                    obuf[0] = off0 + cnt0
                return 0
            lax.fori_loop(0, C // 16, fstep, 0)

            nh0 = obuf[0]
            nh16 = jnp.full((16,), nh0, jnp.int32)
            nsb = (nh0 + HB - 1) // HB

            def build_gq(b, gqx):
                for g in range(HB // 16):
                    gqx[pl.ds(16 * g, 16)] = hitg[pl.ds(b * HB + 16 * g, 16)]

            def build_dq(b):
                for g in range(HB // 16):
                    valid = (b * HB + 16 * g + iota) < nh16
                    d16 = hitd[pl.ds(b * HB + 16 * g, 16)]
                    dq[pl.ds(16 * g, 16)] = jnp.where(valid, d16 - lo, PR)

            def issue(b, gqx, rowsx, sem):
                build_gq(b, gqx)
                pltpu.async_copy(tab.at[gqx], rowsx, sem)

            def drain(rowsx, sem):
                pltpu.make_async_copy(tab.at[pl.ds(0, HB)], rowsx, sem).wait()

            issue(0, gq, rows, gsem)

            @pl.loop(0, nsb // 2)
            def sb_pair(q):
                b0 = 2 * q
                issue(b0 + 1, gq1, rows1, gsem1)
                drain(rows, gsem)
                build_dq(b0)
                pltpu.sync_copy(rows, acc.at[dq], add=True)
                issue(b0 + 2, gq, rows, gsem)
                drain(rows1, gsem1)
                build_dq(b0 + 1)
                pltpu.sync_copy(rows1, acc.at[dq], add=True)

            drain(rows, gsem)
            build_dq(2 * (nsb // 2))
            pltpu.sync_copy(rows, acc.at[dq], add=True)
            return 0
        lax.fori_loop(0, NCHUNK, one_chunk, 0)
        plsc.subcore_barrier()

        for t in range(TPR // HB):
            pltpu.sync_copy(acc.at[pl.ds(base + t * HB, HB)],
                            out.at[pl.ds(lo + base + t * HB, HB)])
        if rem:
            pltpu.sync_copy(
                acc.at[pl.ds(base + (TPR // HB) * HB, rem)],
                out.at[pl.ds(lo + base + (TPR // HB) * HB, rem)])
        plsc.subcore_barrier()
        return 0
    lax.fori_loop(0, NPASS, one_pass, 0)


def _make_scatter_kernel():
    mesh = plsc.VectorSubcoreMesh(core_axis_name="c", subcore_axis_name="s")
    scratch = [
        pltpu.VMEM_SHARED((PR + 8, D), jnp.float32),
        pltpu.VMEM((C,), jnp.int32),
        pltpu.VMEM((C,), jnp.int32),
        pltpu.VMEM((C + 2 * HB,), jnp.int32),
        pltpu.VMEM((C + 2 * HB,), jnp.int32),
        pltpu.VMEM((HB,), jnp.int32),
        pltpu.VMEM((HB,), jnp.int32),
        pltpu.VMEM((HB,), jnp.int32),
        pltpu.VMEM((HB, D), jnp.float32),
        pltpu.VMEM((HB, D), jnp.float32),
        pltpu.VMEM((HB, D), jnp.float32),
        pltpu.SMEM((1,), jnp.int32),
        pltpu.SemaphoreType.DMA,
        pltpu.SemaphoreType.DMA,
    ]
    return pl.kernel(
        _scatter_kernel_body,
        out_type=jax.ShapeDtypeStruct((OUT_ROWS, D), jnp.float32),
        mesh=mesh,
        scratch_types=scratch,
    )


_scatter_kernel = _make_scatter_kernel()


_BLK = 640


def _gemm_block(h_ref, w_ref, o_ref):
    h = jnp.maximum(h_ref[...], 0.0)
    o_ref[...] = jnp.dot(h, w_ref[...], preferred_element_type=jnp.float32)


def _node_transform(new_h, wcat):
    return pl.pallas_call(
        _gemm_block,
        grid=(OUT_ROWS // _BLK,),
        in_specs=[
            pl.BlockSpec((_BLK, D), lambda i: (i, 0)),
            pl.BlockSpec((D, GP), lambda i: (0, 0)),
        ],
        out_specs=pl.BlockSpec((_BLK, GP), lambda i: (i, 0)),
        out_shape=jax.ShapeDtypeStruct((OUT_ROWS, GP), jnp.float32),
    )(new_h, wcat)


_CBLK = 1280


def _cls_block(h_ref, w_ref, b_ref, o_ref):
    h = jnp.maximum(h_ref[...], 0.0)
    o_ref[...] = (jnp.dot(h, w_ref[...], preferred_element_type=jnp.float32)
                  [:, 0] + b_ref[0]).reshape(1, 1, _CBLK)


def _classifier(new_h, w_cls, b_cls):
    out = pl.pallas_call(
        _cls_block,
        grid=(OUT_ROWS // _CBLK,),
        in_specs=[
            pl.BlockSpec((_CBLK, D), lambda i: (i, 0)),
            pl.BlockSpec((D, 1), lambda i: (0, 0)),
            pl.BlockSpec(memory_space=pltpu.SMEM),
        ],
        out_specs=pl.BlockSpec((1, 1, _CBLK), lambda i: (i, 0, 0)),
        out_shape=jax.ShapeDtypeStruct((OUT_ROWS // _CBLK, 1, _CBLK),
                                       jnp.float32),
    )(new_h, w_cls.reshape(D, 1), b_cls.reshape(1))
    return out.reshape(OUT_ROWS)


def kernel(rel_idx, batch_idx, src_idx, rel_e, dst_idx,
           rel_tables, attn1, attn2, W_past, w_cls, b_cls):
    batch_idx = batch_idx.astype(jnp.int32)
    rel_e = rel_e.astype(jnp.int32)
    flat_src = batch_idx * N + src_idx.astype(jnp.int32)
    flat_dst = batch_idx * N + dst_idx.astype(jnp.int32)
    eid = jnp.arange(E, dtype=jnp.int32)

    A1h = attn1[:, :, :D]
    A1r = attn1[:, :, D:2 * D]
    A1q = attn1[:, :, 2 * D:]

    def tables(i):
        Tr = rel_tables[i] @ W_past.T
        Ar = rel_tables[i] @ A1r[i].T
        Aq = rel_tables[i][rel_idx] @ A1q[i].T
        return Tr, Ar, Aq

    Tr0, Ar0, Aq0 = tables(0)
    s0 = jax.nn.sigmoid(
        jnp.maximum(Aq0[:, None, :] + Ar0[None, :, :], 0.0) @ attn2[0].T)
    M0 = (s0 * Tr0[None]).reshape(B * NRELT, D)
    gidx0 = batch_idx * NRELT + rel_e
    new_h = _scatter_kernel(M0, gidx0, flat_dst)

    for i in range(1, L):
        Tr, Ar, Aq = tables(i)
        rtab = jnp.concatenate([Tr, Ar], axis=1)
        wcat = jnp.concatenate(
            [W_past.T, A1h[i].T, jnp.zeros((D, GP - GW), jnp.float32)],
            axis=1)
        G = _node_transform(new_h, wcat)
        msgs = _score_kernel(G, rtab, Aq, attn2[i].reshape(ATTN),
                             rel_e * 8 + batch_idx, flat_src)
        new_h = _scatter_kernel(msgs, eid, flat_dst)

    logits = _classifier(new_h, w_cls, b_cls)
    result = logits[:_ROWS].reshape(B, N)
    probs = jax.nn.softmax(result, axis=1)
    return result, probs


# speedup vs baseline: 1.1072x; 1.1072x over previous
import functools

import jax
import jax.numpy as jnp
from jax import lax
from jax.experimental import pallas as pl
from jax.experimental.pallas import tpu as pltpu
from jax.experimental.pallas import tpu_sc as plsc

B = 8
N = 10000
NREL = 201
NRELT = NREL + 1
D = 128
ATTN = 64
L = 3
E = 320000

_ROWS = B * N
NC = 2
NS = 16
NW = NC * NS
NPASS = 4
PR = 10240
OUT_ROWS = NC * NPASS * PR
TPR = PR // NS
C = 2000
NCHUNK = E // (NS * C)
HB = 96
HS = 80
GW = D + ATTN
GP = 2 * D

_GDN = lax.GatherDimensionNumbers(
    offset_dims=(), collapsed_slice_dims=(0,), start_index_map=(0,))


def _dyn_gather(x, idx):
    return lax.gather(x, idx[:, None], _GDN, (1,),
                      mode=lax.GatherScatterMode.PROMISE_IN_BOUNDS)


def _prefix16(x, iota):
    for sh in (1, 2, 4, 8):
        shifted = _dyn_gather(x, jnp.maximum(iota - sh, 0))
        x = x + jnp.where(iota >= sh, shifted, 0)
    return x


def _sum16(x, iota):
    for sh in (1, 2, 4, 8):
        x = x + _dyn_gather(x, iota ^ sh)
    return x


_NB = E // NW // HS


def _score_kernel_body(tab, trq, a2t, relb, gidx, msgout,
                       rows0, rows1, trq0, trq1, msg, a2_v,
                       relbv, srcv, sem0, sem1, tsem0, tsem1):
    c = lax.axis_index("c")
    s = lax.axis_index("s")
    iota = lax.iota(jnp.int32, 16)

    pltpu.sync_copy(a2t, a2_v)
    w = s * NC + c
    base_e = w * (E // NW)
    pltpu.sync_copy(relb.at[pl.ds(base_e, E // NW)], relbv)
    pltpu.sync_copy(gidx.at[pl.ds(base_e, E // NW)], srcv)

    def compute(sb, rows, trqx):
        a2s = [a2_v[pl.ds(16 * k, 16)] for k in range(ATTN // 16)]

        def one_group(g, _):
            for l in range(16):
                row = 16 * g + l
                acc = jnp.zeros((16,), jnp.float32)
                for k in range(ATTN // 16):
                    t = (rows[row, pl.ds(D + 16 * k, 16)]
                         + trqx[row, pl.ds(D + 16 * k, 16)])
                    acc = acc + jnp.maximum(t, 0.0) * a2s[k]
                tot = _sum16(acc, iota)
                sv = 1.0 / (1.0 + jnp.exp(-tot))
                for k in range(D // 16):
                    msg[row, pl.ds(16 * k, 16)] = (
                        rows[row, pl.ds(16 * k, 16)]
                        + trqx[row, pl.ds(16 * k, 16)]) * sv
            return 0
        lax.fori_loop(0, HS // 16, one_group, 0)
        pltpu.sync_copy(msg, msgout.at[pl.ds(base_e + sb * HS, HS)])

    def issue(sb, rows, sem, trqx, tsem):
        pltpu.async_copy(tab.at[srcv.at[pl.ds(sb * HS, HS)]], rows, sem)
        pltpu.async_copy(trq.at[relbv.at[pl.ds(sb * HS, HS)]], trqx, tsem)

    def drain(rows, sem, trqx, tsem):
        pltpu.make_async_copy(tab.at[pl.ds(0, HS)], rows, sem).wait()
        pltpu.make_async_copy(trq.at[pl.ds(0, HS)], trqx, tsem).wait()

    issue(0, rows0, sem0, trq0, tsem0)

    def pair(q, _):
        b0 = 2 * q
        issue(b0 + 1, rows1, sem1, trq1, tsem1)
        drain(rows0, sem0, trq0, tsem0)
        compute(b0, rows0, trq0)
        issue(b0 + 2, rows0, sem0, trq0, tsem0)
        drain(rows1, sem1, trq1, tsem1)
        compute(b0 + 1, rows1, trq1)
        return 0
    lax.fori_loop(0, (_NB - 1) // 2, pair, 0)
    drain(rows0, sem0, trq0, tsem0)
    compute(_NB - 1, rows0, trq0)


def _make_score_kernel():
    mesh = plsc.VectorSubcoreMesh(core_axis_name="c", subcore_axis_name="s")
    scratch = [
        pltpu.VMEM((HS, GP), jnp.float32),
        pltpu.VMEM((HS, GP), jnp.float32),
        pltpu.VMEM((HS, GP), jnp.float32),
        pltpu.VMEM((HS, GP), jnp.float32),
        pltpu.VMEM((HS, D), jnp.float32),
        pltpu.VMEM((ATTN,), jnp.float32),
        pltpu.VMEM((E // NW,), jnp.int32),
        pltpu.VMEM((E // NW,), jnp.int32),
        pltpu.SemaphoreType.DMA,
        pltpu.SemaphoreType.DMA,
        pltpu.SemaphoreType.DMA,
        pltpu.SemaphoreType.DMA,
    ]
    return pl.kernel(
        _score_kernel_body,
        out_type=jax.ShapeDtypeStruct((E, D), jnp.float32),
        mesh=mesh,
        scratch_types=scratch,
    )


_score_kernel = _make_score_kernel()


def _scatter_kernel_body(tab, gidx, dst, out,
                         acc, dstv, gidxv, hitg, hitd, gq, gq1, dq,
                         rows, rows1, zrow, obuf, gsem, gsem1):
    c = lax.axis_index("c")
    s = lax.axis_index("s")
    iota = lax.iota(jnp.int32, 16)
    zf16 = jnp.zeros((16,), jnp.float32)
    zi16 = jnp.zeros((16,), jnp.int32)

    def _zr(i, _):
        for k in range(D // 16):
            zrow[i, pl.ds(16 * k, 16)] = zf16
        return 0
    lax.fori_loop(0, HB, _zr, 0)

    def _zh(k, _):
        hitg[pl.ds(16 * k, 16)] = zi16
        hitd[pl.ds(16 * k, 16)] = zi16
        return 0
    lax.fori_loop(0, (C + 2 * HB) // 16, _zh, 0)

    def one_pass(p, _):
        lo = (c * NPASS + p) * PR
        hi = lo + PR

        base = s * TPR
        for t in range(TPR // HB):
            pltpu.sync_copy(zrow, acc.at[pl.ds(base + t * HB, HB)])
        rem = TPR % HB
        if rem:
            pltpu.sync_copy(zrow.at[pl.ds(0, rem)],
                            acc.at[pl.ds(base + (TPR // HB) * HB, rem)])
        plsc.subcore_barrier()

        def one_chunk(k, _):
            ebase = s * (NCHUNK * C) + k * C
            pltpu.sync_copy(dst.at[pl.ds(ebase, C)], dstv)
            pltpu.sync_copy(gidx.at[pl.ds(ebase, C)], gidxv)
            obuf[0] = 0

            def fstep(g, _):
                d16 = dstv[pl.ds(16 * g, 16)]
                m = (d16 >= lo) & (d16 < hi)
                mi = jnp.where(m, 1, 0)
                cs = _prefix16(mi, iota)
                cnt0 = cs[15]

                @pl.when(cnt0 > 0)
                def _():
                    g16 = gidxv[pl.ds(16 * g, 16)]
                    tgt = iota + 1
                    inv = jnp.zeros((16,), jnp.int32)
                    for step in (8, 4, 2, 1):
                        cand = inv + step
                        probe = _dyn_gather(cs, cand - 1)
                        inv = jnp.where(probe < tgt, cand, inv)
                    gc = _dyn_gather(g16, jnp.minimum(inv, 15))
                    dc = _dyn_gather(d16, jnp.minimum(inv, 15))
                    off0 = obuf[0]
                    r = off0 & 15
                    a = off0 - r
                    r16 = jnp.full((16,), r, jnp.int32)
                    perm = (iota - r16) & 15
                    rotg = _dyn_gather(gc, perm)
                    rotd = _dyn_gather(dc, perm)
                    sel0 = iota >= r16
                    b0 = hitg[pl.ds(a, 16)]
                    hitg[pl.ds(a, 16)] = jnp.where(sel0, rotg, b0)
                    b0d = hitd[pl.ds(a, 16)]
                    hitd[pl.ds(a, 16)] = jnp.where(sel0, rotd, b0d)
                    b1 = hitg[pl.ds(a + 16, 16)]
                    hitg[pl.ds(a + 16, 16)] = jnp.where(sel0, b1, rotg)
                    b1d = hitd[pl.ds(a + 16, 16)]
                    hitd[pl.ds(a + 16, 16)] = jnp.where(sel0, b1d, rotd)
                    obuf[0] = off0 + cnt0
                return 0
            lax.fori_loop(0, C // 16, fstep, 0)

            nh0 = obuf[0]
            nh16 = jnp.full((16,), nh0, jnp.int32)
            nsb = (nh0 + HB - 1) // HB

            def build_gq(b, gqx):
                for g in range(HB // 16):
                    gqx[pl.ds(16 * g, 16)] = hitg[pl.ds(b * HB + 16 * g, 16)]

            def build_dq(b):
                for g in range(HB // 16):
                    valid = (b * HB + 16 * g + iota) < nh16
                    d16 = hitd[pl.ds(b * HB + 16 * g, 16)]
                    dq[pl.ds(16 * g, 16)] = jnp.where(valid, d16 - lo, PR)

            def issue(b, gqx, rowsx, sem):
                build_gq(b, gqx)
                pltpu.async_copy(tab.at[gqx], rowsx, sem)

            def drain(rowsx, sem):
                pltpu.make_async_copy(tab.at[pl.ds(0, HB)], rowsx, sem).wait()

            issue(0, gq, rows, gsem)

            @pl.loop(0, nsb // 2)
            def sb_pair(q):
                b0 = 2 * q
                issue(b0 + 1, gq1, rows1, gsem1)
                drain(rows, gsem)
                build_dq(b0)
                pltpu.sync_copy(rows, acc.at[dq], add=True)
                issue(b0 + 2, gq, rows, gsem)
                drain(rows1, gsem1)
                build_dq(b0 + 1)
                pltpu.sync_copy(rows1, acc.at[dq], add=True)

            drain(rows, gsem)
            build_dq(2 * (nsb // 2))
            pltpu.sync_copy(rows, acc.at[dq], add=True)
            return 0
        lax.fori_loop(0, NCHUNK, one_chunk, 0)
        plsc.subcore_barrier()

        for t in range(TPR // HB):
            pltpu.sync_copy(acc.at[pl.ds(base + t * HB, HB)],
                            out.at[pl.ds(lo + base + t * HB, HB)])
        if rem:
            pltpu.sync_copy(
                acc.at[pl.ds(base + (TPR // HB) * HB, rem)],
                out.at[pl.ds(lo + base + (TPR // HB) * HB, rem)])
        plsc.subcore_barrier()
        return 0
    lax.fori_loop(0, NPASS, one_pass, 0)


def _make_scatter_kernel():
    mesh = plsc.VectorSubcoreMesh(core_axis_name="c", subcore_axis_name="s")
    scratch = [
        pltpu.VMEM_SHARED((PR + 8, D), jnp.float32),
        pltpu.VMEM((C,), jnp.int32),
        pltpu.VMEM((C,), jnp.int32),
        pltpu.VMEM((C + 2 * HB,), jnp.int32),
        pltpu.VMEM((C + 2 * HB,), jnp.int32),
        pltpu.VMEM((HB,), jnp.int32),
        pltpu.VMEM((HB,), jnp.int32),
        pltpu.VMEM((HB,), jnp.int32),
        pltpu.VMEM((HB, D), jnp.float32),
        pltpu.VMEM((HB, D), jnp.float32),
        pltpu.VMEM((HB, D), jnp.float32),
        pltpu.SMEM((1,), jnp.int32),
        pltpu.SemaphoreType.DMA,
        pltpu.SemaphoreType.DMA,
    ]
    return pl.kernel(
        _scatter_kernel_body,
        out_type=jax.ShapeDtypeStruct((OUT_ROWS, D), jnp.float32),
        mesh=mesh,
        scratch_types=scratch,
    )


_scatter_kernel = _make_scatter_kernel()


_BLK = 640


def _gemm_block(h_ref, w_ref, o_ref):
    h = jnp.maximum(h_ref[...], 0.0)
    o_ref[...] = jnp.dot(h, w_ref[...], preferred_element_type=jnp.float32)


def _node_transform(new_h, wcat):
    return pl.pallas_call(
        _gemm_block,
        grid=(OUT_ROWS // _BLK,),
        in_specs=[
            pl.BlockSpec((_BLK, D), lambda i: (i, 0)),
            pl.BlockSpec((D, GP), lambda i: (0, 0)),
        ],
        out_specs=pl.BlockSpec((_BLK, GP), lambda i: (i, 0)),
        out_shape=jax.ShapeDtypeStruct((OUT_ROWS, GP), jnp.float32),
    )(new_h, wcat)


_CBLK = 1280


def _cls_block(h_ref, w_ref, b_ref, o_ref):
    h = jnp.maximum(h_ref[...], 0.0)
    o_ref[...] = (jnp.dot(h, w_ref[...], preferred_element_type=jnp.float32)
                  [:, 0] + b_ref[0]).reshape(1, 1, _CBLK)


def _classifier(new_h, w_cls, b_cls):
    out = pl.pallas_call(
        _cls_block,
        grid=(OUT_ROWS // _CBLK,),
        in_specs=[
            pl.BlockSpec((_CBLK, D), lambda i: (i, 0)),
            pl.BlockSpec((D, 1), lambda i: (0, 0)),
            pl.BlockSpec(memory_space=pltpu.SMEM),
        ],
        out_specs=pl.BlockSpec((1, 1, _CBLK), lambda i: (i, 0, 0)),
        out_shape=jax.ShapeDtypeStruct((OUT_ROWS // _CBLK, 1, _CBLK),
                                       jnp.float32),
    )(new_h, w_cls.reshape(D, 1), b_cls.reshape(1))
    return out.reshape(OUT_ROWS)


def kernel(rel_idx, batch_idx, src_idx, rel_e, dst_idx,
           rel_tables, attn1, attn2, W_past, w_cls, b_cls):
    batch_idx = batch_idx.astype(jnp.int32)
    rel_e = rel_e.astype(jnp.int32)
    flat_src = batch_idx * N + src_idx.astype(jnp.int32)
    flat_dst = batch_idx * N + dst_idx.astype(jnp.int32)
    eid = jnp.arange(E, dtype=jnp.int32)

    A1h = attn1[:, :, :D]
    A1r = attn1[:, :, D:2 * D]
    A1q = attn1[:, :, 2 * D:]

    def tables(i):
        Tr = rel_tables[i] @ W_past.T
        Ar = rel_tables[i] @ A1r[i].T
        Aq = rel_tables[i][rel_idx] @ A1q[i].T
        return Tr, Ar, Aq

    Tr0, Ar0, Aq0 = tables(0)
    s0 = jax.nn.sigmoid(
        jnp.maximum(Aq0[:, None, :] + Ar0[None, :, :], 0.0) @ attn2[0].T)
    M0 = (s0 * Tr0[None]).reshape(B * NRELT, D)
    gidx0 = batch_idx * NRELT + rel_e
    new_h = _scatter_kernel(M0, gidx0, flat_dst)

    for i in range(1, L):
        Tr, Ar, Aq = tables(i)
        trq = jnp.concatenate(
            [jnp.broadcast_to(Tr[:, None, :], (NRELT, B, D)),
             Ar[:, None, :] + Aq[None, :, :],
             jnp.zeros((NRELT, B, GP - GW), jnp.float32)],
            axis=2).reshape(NRELT * B, GP)
        wcat = jnp.concatenate(
            [W_past.T, A1h[i].T, jnp.zeros((D, GP - GW), jnp.float32)],
            axis=1)
        G = _node_transform(new_h, wcat)
        msgs = _score_kernel(G, trq, attn2[i].reshape(ATTN),
                             rel_e * 8 + batch_idx, flat_src)
        new_h = _scatter_kernel(msgs, eid, flat_dst)

    logits = _classifier(new_h, w_cls, b_cls)
    result = logits[:_ROWS].reshape(B, N)
    probs = jax.nn.softmax(result, axis=1)
    return result, probs


# speedup vs baseline: 1.1429x; 1.0322x over previous
import functools

import jax
import jax.numpy as jnp
from jax import lax
from jax.experimental import pallas as pl
from jax.experimental.pallas import tpu as pltpu
from jax.experimental.pallas import tpu_sc as plsc

B = 8
N = 10000
NREL = 201
NRELT = NREL + 1
D = 128
ATTN = 64
L = 3
E = 320000

_ROWS = B * N
NC = 2
NS = 16
NW = NC * NS
NPASS = 4
PR = 10240
OUT_ROWS = NC * NPASS * PR
TPR = PR // NS
C = 2000
NCHUNK = E // (NS * C)
HB = 96
HS = 80
GW = D + ATTN
GP = 2 * D

_GDN = lax.GatherDimensionNumbers(
    offset_dims=(), collapsed_slice_dims=(0,), start_index_map=(0,))


def _dyn_gather(x, idx):
    return lax.gather(x, idx[:, None], _GDN, (1,),
                      mode=lax.GatherScatterMode.PROMISE_IN_BOUNDS)


def _prefix16(x, iota):
    for sh in (1, 2, 4, 8):
        shifted = _dyn_gather(x, jnp.maximum(iota - sh, 0))
        x = x + jnp.where(iota >= sh, shifted, 0)
    return x


def _sum16(x, iota):
    for sh in (1, 2, 4, 8):
        x = x + _dyn_gather(x, iota ^ sh)
    return x


_NB = E // NW // HS


def _score_kernel_body(tab, trq, a2t, relb, gidx, msgout,
                       rows0, rows1, trq0, trq1, msg, a2_v,
                       relbv, srcv, sem0, sem1, tsem0, tsem1):
    c = lax.axis_index("c")
    s = lax.axis_index("s")
    iota = lax.iota(jnp.int32, 16)

    pltpu.sync_copy(a2t, a2_v)
    w = s * NC + c
    base_e = w * (E // NW)
    pltpu.sync_copy(relb.at[pl.ds(base_e, E // NW)], relbv)
    pltpu.sync_copy(gidx.at[pl.ds(base_e, E // NW)], srcv)

    def compute(sb, rows, trqx):
        a2s = [a2_v[pl.ds(16 * k, 16)] for k in range(ATTN // 16)]

        def one_group(g, _):
            for l in range(16):
                row = 16 * g + l
                acc = jnp.zeros((16,), jnp.float32)
                for k in range(ATTN // 16):
                    t = (rows[row, pl.ds(D + 16 * k, 16)]
                         + trqx[row, pl.ds(D + 16 * k, 16)])
                    acc = acc + jnp.maximum(t, 0.0) * a2s[k]
                tot = _sum16(acc, iota)
                sv = 1.0 / (1.0 + jnp.exp(-tot))
                for k in range(D // 16):
                    msg[row, pl.ds(16 * k, 16)] = (
                        rows[row, pl.ds(16 * k, 16)]
                        + trqx[row, pl.ds(16 * k, 16)]) * sv
            return 0
        lax.fori_loop(0, HS // 16, one_group, 0)
        pltpu.sync_copy(msg, msgout.at[pl.ds(base_e + sb * HS, HS)])

    def issue(sb, rows, sem, trqx, tsem):
        pltpu.async_copy(tab.at[srcv.at[pl.ds(sb * HS, HS)]], rows, sem)
        pltpu.async_copy(trq.at[relbv.at[pl.ds(sb * HS, HS)]], trqx, tsem)

    def drain(rows, sem, trqx, tsem):
        pltpu.make_async_copy(tab.at[pl.ds(0, HS)], rows, sem).wait()
        pltpu.make_async_copy(trq.at[pl.ds(0, HS)], trqx, tsem).wait()

    issue(0, rows0, sem0, trq0, tsem0)

    def pair(q, _):
        b0 = 2 * q
        issue(b0 + 1, rows1, sem1, trq1, tsem1)
        drain(rows0, sem0, trq0, tsem0)
        compute(b0, rows0, trq0)
        issue(b0 + 2, rows0, sem0, trq0, tsem0)
        drain(rows1, sem1, trq1, tsem1)
        compute(b0 + 1, rows1, trq1)
        return 0
    lax.fori_loop(0, (_NB - 1) // 2, pair, 0)
    drain(rows0, sem0, trq0, tsem0)
    compute(_NB - 1, rows0, trq0)


def _make_score_kernel():
    mesh = plsc.VectorSubcoreMesh(core_axis_name="c", subcore_axis_name="s")
    scratch = [
        pltpu.VMEM((HS, GP), jnp.float32),
        pltpu.VMEM((HS, GP), jnp.float32),
        pltpu.VMEM((HS, GP), jnp.float32),
        pltpu.VMEM((HS, GP), jnp.float32),
        pltpu.VMEM((HS, D), jnp.float32),
        pltpu.VMEM((ATTN,), jnp.float32),
        pltpu.VMEM((E // NW,), jnp.int32),
        pltpu.VMEM((E // NW,), jnp.int32),
        pltpu.SemaphoreType.DMA,
        pltpu.SemaphoreType.DMA,
        pltpu.SemaphoreType.DMA,
        pltpu.SemaphoreType.DMA,
    ]
    return pl.kernel(
        _score_kernel_body,
        out_type=jax.ShapeDtypeStruct((E, D), jnp.float32),
        mesh=mesh,
        scratch_types=scratch,
    )


_score_kernel = _make_score_kernel()


def _scatter_kernel_body(tab, gidx, dst, out,
                         acc, dstv, gidxv, dstv1, gidxv1, hitg, hitd,
                         gq, gq1, dq, rows, rows1, zrow, obuf,
                         gsem, gsem1, fsem, fsem1):
    c = lax.axis_index("c")
    s = lax.axis_index("s")
    iota = lax.iota(jnp.int32, 16)
    zf16 = jnp.zeros((16,), jnp.float32)
    zi16 = jnp.zeros((16,), jnp.int32)

    def _zr(i, _):
        for k in range(D // 16):
            zrow[i, pl.ds(16 * k, 16)] = zf16
        return 0
    lax.fori_loop(0, 64, _zr, 0)

    def _zh(k, _):
        hitg[pl.ds(16 * k, 16)] = zi16
        hitd[pl.ds(16 * k, 16)] = zi16
        return 0
    lax.fori_loop(0, (C + 2 * HB) // 16, _zh, 0)

    def one_pass(p, _):
        lo = (c * NPASS + p) * PR
        hi = lo + PR

        base = s * TPR
        for t in range(TPR // 64):
            pltpu.sync_copy(zrow, acc.at[pl.ds(base + t * 64, 64)])
        plsc.subcore_barrier()

        def issue_flds(k, dv, gv, fs):
            eb = s * (NCHUNK * C) + jnp.minimum(k, NCHUNK - 1) * C
            pltpu.async_copy(dst.at[pl.ds(eb, C)], dv, fs)
            pltpu.async_copy(gidx.at[pl.ds(eb, C)], gv, fs)

        def drain_flds(dv, gv, fs):
            pltpu.make_async_copy(dst.at[pl.ds(0, C)], dv, fs).wait()
            pltpu.make_async_copy(gidx.at[pl.ds(0, C)], gv, fs).wait()

        def process(dv, gv):
            obuf[0] = 0

            def fstep(g, _):
                d16 = dv[pl.ds(16 * g, 16)]
                m = (d16 >= lo) & (d16 < hi)
                mi = jnp.where(m, 1, 0)
                cs = _prefix16(mi, iota)
                cnt0 = cs[15]

                @pl.when(cnt0 > 0)
                def _():
                    g16 = gv[pl.ds(16 * g, 16)]
                    tgt = iota + 1
                    inv = jnp.zeros((16,), jnp.int32)
                    for step in (8, 4, 2, 1):
                        cand = inv + step
                        probe = _dyn_gather(cs, cand - 1)
                        inv = jnp.where(probe < tgt, cand, inv)
                    gc = _dyn_gather(g16, jnp.minimum(inv, 15))
                    dc = _dyn_gather(d16, jnp.minimum(inv, 15))
                    off0 = obuf[0]
                    r = off0 & 15
                    a = off0 - r
                    r16 = jnp.full((16,), r, jnp.int32)
                    perm = (iota - r16) & 15
                    rotg = _dyn_gather(gc, perm)
                    rotd = _dyn_gather(dc, perm)
                    sel0 = iota >= r16
                    b0 = hitg[pl.ds(a, 16)]
                    hitg[pl.ds(a, 16)] = jnp.where(sel0, rotg, b0)
                    b0d = hitd[pl.ds(a, 16)]
                    hitd[pl.ds(a, 16)] = jnp.where(sel0, rotd, b0d)
                    b1 = hitg[pl.ds(a + 16, 16)]
                    hitg[pl.ds(a + 16, 16)] = jnp.where(sel0, b1, rotg)
                    b1d = hitd[pl.ds(a + 16, 16)]
                    hitd[pl.ds(a + 16, 16)] = jnp.where(sel0, b1d, rotd)
                    obuf[0] = off0 + cnt0
                return 0
            lax.fori_loop(0, C // 16, fstep, 0)

            nh0 = obuf[0]
            nh16 = jnp.full((16,), nh0, jnp.int32)
            nsb = (nh0 + HB - 1) // HB

            def build_gq(b, gqx):
                for g in range(HB // 16):
                    gqx[pl.ds(16 * g, 16)] = hitg[pl.ds(b * HB + 16 * g, 16)]

            def build_dq(b):
                for g in range(HB // 16):
                    valid = (b * HB + 16 * g + iota) < nh16
                    d16 = hitd[pl.ds(b * HB + 16 * g, 16)]
                    dq[pl.ds(16 * g, 16)] = jnp.where(valid, d16 - lo, PR)

            def issue(b, gqx, rowsx, sem):
                build_gq(b, gqx)
                pltpu.async_copy(tab.at[gqx], rowsx, sem)

            def drain(rowsx, sem):
                pltpu.make_async_copy(tab.at[pl.ds(0, HB)], rowsx, sem).wait()

            issue(0, gq, rows, gsem)

            @pl.loop(0, nsb // 2)
            def sb_pair(q):
                b0 = 2 * q
                issue(b0 + 1, gq1, rows1, gsem1)
                drain(rows, gsem)
                build_dq(b0)
                pltpu.sync_copy(rows, acc.at[dq], add=True)
                issue(b0 + 2, gq, rows, gsem)
                drain(rows1, gsem1)
                build_dq(b0 + 1)
                pltpu.sync_copy(rows1, acc.at[dq], add=True)

            drain(rows, gsem)
            build_dq(2 * (nsb // 2))
            pltpu.sync_copy(rows, acc.at[dq], add=True)

        issue_flds(0, dstv, gidxv, fsem)

        @pl.loop(0, NCHUNK // 2)
        def chunk_pair(q):
            k0 = 2 * q
            issue_flds(k0 + 1, dstv1, gidxv1, fsem1)
            drain_flds(dstv, gidxv, fsem)
            process(dstv, gidxv)
            issue_flds(k0 + 2, dstv, gidxv, fsem)
            drain_flds(dstv1, gidxv1, fsem1)
            process(dstv1, gidxv1)

        drain_flds(dstv, gidxv, fsem)
        plsc.subcore_barrier()

        for t in range(TPR // 64):
            pltpu.sync_copy(acc.at[pl.ds(base + t * 64, 64)],
                            out.at[pl.ds(lo + base + t * 64, 64)])
        plsc.subcore_barrier()
        return 0
    lax.fori_loop(0, NPASS, one_pass, 0)


def _make_scatter_kernel():
    mesh = plsc.VectorSubcoreMesh(core_axis_name="c", subcore_axis_name="s")
    scratch = [
        pltpu.VMEM_SHARED((PR + 8, D), jnp.float32),
        pltpu.VMEM((C,), jnp.int32),
        pltpu.VMEM((C,), jnp.int32),
        pltpu.VMEM((C,), jnp.int32),
        pltpu.VMEM((C,), jnp.int32),
        pltpu.VMEM((C + 2 * HB,), jnp.int32),
        pltpu.VMEM((C + 2 * HB,), jnp.int32),
        pltpu.VMEM((HB,), jnp.int32),
        pltpu.VMEM((HB,), jnp.int32),
        pltpu.VMEM((HB,), jnp.int32),
        pltpu.VMEM((HB, D), jnp.float32),
        pltpu.VMEM((HB, D), jnp.float32),
        pltpu.VMEM((64, D), jnp.float32),
        pltpu.SMEM((1,), jnp.int32),
        pltpu.SemaphoreType.DMA,
        pltpu.SemaphoreType.DMA,
        pltpu.SemaphoreType.DMA,
        pltpu.SemaphoreType.DMA,
    ]
    return pl.kernel(
        _scatter_kernel_body,
        out_type=jax.ShapeDtypeStruct((OUT_ROWS, D), jnp.float32),
        mesh=mesh,
        scratch_types=scratch,
    )


_scatter_kernel = _make_scatter_kernel()


_BLK = 640


def _gemm_block(h_ref, w_ref, o_ref):
    h = jnp.maximum(h_ref[...], 0.0)
    o_ref[...] = jnp.dot(h, w_ref[...], preferred_element_type=jnp.float32)


def _node_transform(new_h, wcat):
    return pl.pallas_call(
        _gemm_block,
        grid=(OUT_ROWS // _BLK,),
        in_specs=[
            pl.BlockSpec((_BLK, D), lambda i: (i, 0)),
            pl.BlockSpec((D, GP), lambda i: (0, 0)),
        ],
        out_specs=pl.BlockSpec((_BLK, GP), lambda i: (i, 0)),
        out_shape=jax.ShapeDtypeStruct((OUT_ROWS, GP), jnp.float32),
    )(new_h, wcat)


_CBLK = 1280


def _cls_block(h_ref, w_ref, b_ref, o_ref):
    h = jnp.maximum(h_ref[...], 0.0)
    o_ref[...] = (jnp.dot(h, w_ref[...], preferred_element_type=jnp.float32)
                  [:, 0] + b_ref[0]).reshape(1, 1, _CBLK)


def _classifier(new_h, w_cls, b_cls):
    out = pl.pallas_call(
        _cls_block,
        grid=(OUT_ROWS // _CBLK,),
        in_specs=[
            pl.BlockSpec((_CBLK, D), lambda i: (i, 0)),
            pl.BlockSpec((D, 1), lambda i: (0, 0)),
            pl.BlockSpec(memory_space=pltpu.SMEM),
        ],
        out_specs=pl.BlockSpec((1, 1, _CBLK), lambda i: (i, 0, 0)),
        out_shape=jax.ShapeDtypeStruct((OUT_ROWS // _CBLK, 1, _CBLK),
                                       jnp.float32),
    )(new_h, w_cls.reshape(D, 1), b_cls.reshape(1))
    return out.reshape(OUT_ROWS)


def kernel(rel_idx, batch_idx, src_idx, rel_e, dst_idx,
           rel_tables, attn1, attn2, W_past, w_cls, b_cls):
    batch_idx = batch_idx.astype(jnp.int32)
    rel_e = rel_e.astype(jnp.int32)
    flat_src = batch_idx * N + src_idx.astype(jnp.int32)
    flat_dst = batch_idx * N + dst_idx.astype(jnp.int32)
    eid = jnp.arange(E, dtype=jnp.int32)

    A1h = attn1[:, :, :D]
    A1r = attn1[:, :, D:2 * D]
    A1q = attn1[:, :, 2 * D:]

    def tables(i):
        Tr = rel_tables[i] @ W_past.T
        Ar = rel_tables[i] @ A1r[i].T
        Aq = rel_tables[i][rel_idx] @ A1q[i].T
        return Tr, Ar, Aq

    Tr0, Ar0, Aq0 = tables(0)
    s0 = jax.nn.sigmoid(
        jnp.maximum(Aq0[:, None, :] + Ar0[None, :, :], 0.0) @ attn2[0].T)
    M0 = (s0 * Tr0[None]).reshape(B * NRELT, D)
    gidx0 = batch_idx * NRELT + rel_e
    new_h = _scatter_kernel(M0, gidx0, flat_dst)

    for i in range(1, L):
        Tr, Ar, Aq = tables(i)
        trq = jnp.concatenate(
            [jnp.broadcast_to(Tr[:, None, :], (NRELT, B, D)),
             Ar[:, None, :] + Aq[None, :, :],
             jnp.zeros((NRELT, B, GP - GW), jnp.float32)],
            axis=2).reshape(NRELT * B, GP)
        wcat = jnp.concatenate(
            [W_past.T, A1h[i].T, jnp.zeros((D, GP - GW), jnp.float32)],
            axis=1)
        G = _node_transform(new_h, wcat)
        msgs = _score_kernel(G, trq, attn2[i].reshape(ATTN),
                             rel_e * 8 + batch_idx, flat_src)
        new_h = _scatter_kernel(msgs, eid, flat_dst)

    logits = _classifier(new_h, w_cls, b_cls)
    result = logits[:_ROWS].reshape(B, N)
    probs = jax.nn.softmax(result, axis=1)
    return result, probs
